# baseline (speedup 1.0000x reference)
import functools

import jax
import jax.numpy as jnp
from jax import lax
from jax.experimental import pallas as pl
from jax.experimental.pallas import tpu as pltpu
from jax.experimental.pallas import tpu_sc as plsc

N = 4096
C = 128
K = 16
TM = 256


def _knn_body(sqc_ref, sqr_ref, xb_ref, xa_ref, o_ref, s_ref):
    xb = xb_ref[...]
    xa = xa_ref[...]
    xy = lax.dot_general(xb, xa, (((1,), (1,)), ((), ())),
                         preferred_element_type=jnp.float32)
    d2 = (sqc_ref[...] + sqr_ref[...]) - 2.0 * xy
    i = pl.program_id(0)
    rows = i * TM + lax.broadcasted_iota(jnp.int32, (TM, N), 0)
    cols = lax.broadcasted_iota(jnp.int32, (TM, N), 1)
    d2 = d2 + jnp.where(rows == cols, jnp.float32(1e10), jnp.float32(0.0))
    s_ref[...] = d2

    NRG = TM // 8

    def one_round(t):
        lane32 = lax.broadcasted_iota(jnp.int32, (8, 32), 1)
        for rg in range(NRG):
            sl = slice(rg * 8, rg * 8 + 8)
            S = s_ref[sl, :]
            glane = lax.broadcasted_iota(
                jnp.int32, (8, N), 1).astype(jnp.float32)
            m = jnp.min(S, axis=1, keepdims=True)
            cand = jnp.where(S == m, glane, jnp.float32(1e9))
            a = jnp.min(cand, axis=1, keepdims=True)
            o_ref[sl, :] = jnp.where(lane32 == t, a.astype(jnp.int32),
                                     o_ref[sl, :])
            s_ref[sl, :] = jnp.where(cand == a, jnp.float32(jnp.inf), S)

    def rnd(i, _):
        one_round(2 * i)
        one_round(2 * i + 1)
        return 0

    lax.fori_loop(0, 16, rnd, 0)


def _knn32(x):
    sq = jnp.sum(x * x, axis=1)
    sqc = sq.reshape(N, 1)
    sqr = sq.reshape(1, N)
    return pl.pallas_call(
        _knn_body,
        grid=(N // TM,),
        in_specs=[
            pl.BlockSpec((TM, 1), lambda i: (i, 0)),
            pl.BlockSpec((1, N), lambda i: (0, 0)),
            pl.BlockSpec((TM, C), lambda i: (i, 0)),
            pl.BlockSpec((N, C), lambda i: (0, 0)),
        ],
        out_specs=pl.BlockSpec((TM, 32), lambda i: (i, 0)),
        out_shape=jax.ShapeDtypeStruct((N, 32), jnp.int32),
        scratch_shapes=[pltpu.VMEM((TM, N), jnp.float32)],
    )(sqc, sqr, x, x)


def _leaky(z):
    return jnp.where(z >= 0, z, jnp.float32(0.2) * z)


def _layer_mm(parts, c_prev, gm_prev, W, b):
    n_parts = len(parts)
    has_agg = c_prev is not None
    cin = C * (n_parts + (1 if has_agg else 0))

    def body(*refs):
        part_refs = refs[:n_parts]
        r = n_parts
        if has_agg:
            cp_ref, gm_ref = refs[r], refs[r + 1]
            r += 2
        W_ref, b_ref = refs[r], refs[r + 1]
        outs = refs[r + 2:]
        cols = [pr[...] for pr in part_refs]
        if has_agg:
            agg = _leaky(cp_ref[...] + gm_ref[...])
            cols.append(agg)
        X = jnp.concatenate(cols, axis=1) if len(cols) > 1 else cols[0]
        Wt = W_ref[:cin, :]
        Wb = W_ref[cin:, :]
        hp = lax.Precision.HIGHEST
        bm = lax.dot_general(X, Wb, (((1,), (0,)), ((), ())), precision=hp,
                             preferred_element_type=jnp.float32)
        cc = lax.dot_general(X, Wt - Wb, (((1,), (0,)), ((), ())), precision=hp,
                             preferred_element_type=jnp.float32) + b_ref[...]
        outs[0][...] = bm
        outs[1][...] = cc
        if has_agg:
            outs[2][...] = agg

    inputs = list(parts)
    if has_agg:
        inputs += [c_prev, gm_prev]
    inputs += [W, b.reshape(1, C)]
    out_shape = [jax.ShapeDtypeStruct((N, C), jnp.float32),
                 jax.ShapeDtypeStruct((N, C), jnp.float32)]
    if has_agg:
        out_shape.append(jax.ShapeDtypeStruct((N, C), jnp.float32))
    return pl.pallas_call(body, out_shape=out_shape)(*inputs)


def _gather_max2(bm1, idx1flat, bm2, idx2flat):
    NW = 32
    nodes_w = N // NW
    ch_nodes = 8
    ch_idx = ch_nodes * K
    mesh = plsc.VectorSubcoreMesh(core_axis_name="c", subcore_axis_name="s")

    nch = nodes_w // ch_nodes

    @functools.partial(
        pl.kernel, mesh=mesh,
        out_type=[jax.ShapeDtypeStruct((N, C), jnp.float32),
                  jax.ShapeDtypeStruct((N, C), jnp.float32)],
        scratch_types=[
            pltpu.VMEM((ch_idx,), jnp.int32),
            pltpu.VMEM((ch_idx,), jnp.int32),
            pltpu.VMEM((ch_idx, C), jnp.float32),
            pltpu.VMEM((ch_idx, C), jnp.float32),
            pltpu.VMEM((ch_nodes, C), jnp.float32),
            pltpu.SemaphoreType.DMA,
            pltpu.SemaphoreType.DMA,
        ])
    def k(bm1_hbm, i1_hbm, bm2_hbm, i2_hbm, o1_hbm, o2_hbm,
          ia, ib, ra, rb, out_v, sa, sb):
        wid = lax.axis_index("s") * 2 + lax.axis_index("c")
        base_node = wid * nodes_w

        def branch(bm_hbm, idx_hbm, out_hbm):
            def start(ci, iv, rv, sem):
                pltpu.sync_copy(
                    idx_hbm.at[pl.ds((base_node + ci * ch_nodes) * K, ch_idx)],
                    iv)
                pltpu.async_copy(bm_hbm.at[iv], rv, sem)

            def finish(ci, iv, rv, sem):
                pltpu.make_async_copy(bm_hbm.at[iv], rv, sem).wait()

                @pl.loop(0, ch_nodes)
                def _node(m):
                    for c in range(C // 16):
                        sl = pl.ds(c * 16, 16)
                        acc = rv[m * K, sl]
                        for j in range(1, K):
                            acc = jnp.maximum(acc, rv[m * K + j, sl])
                        out_v[m, sl] = acc

                pltpu.sync_copy(
                    out_v,
                    out_hbm.at[pl.ds(base_node + ci * ch_nodes, ch_nodes)])

            start(0, ia, ra, sa)

            @pl.loop(0, nch // 2 - 1)
            def _g(g):
                start(2 * g + 1, ib, rb, sb)
                finish(2 * g, ia, ra, sa)
                start(2 * g + 2, ia, ra, sa)
                finish(2 * g + 1, ib, rb, sb)

            start(nch - 1, ib, rb, sb)
            finish(nch - 2, ia, ra, sa)
            finish(nch - 1, ib, rb, sb)

        branch(bm1_hbm, i1_hbm, o1_hbm)
        branch(bm2_hbm, i2_hbm, o2_hbm)

    return k(bm1, idx1flat, bm2, idx2flat)


def _final_body(x_ref, a01, a11, c21, g21, a02, a12, c22, g22,
                z0, z1, z2, z3):
    def quadmax(z):
        t = jnp.maximum(z, jnp.concatenate([z[:, 1:], z[:, :1]], axis=1))
        return jnp.maximum(t, jnp.concatenate([t[:, 2:], t[:, :2]], axis=1))

    agg21 = _leaky(c21[...] + g21[...])
    agg22 = _leaky(c22[...] + g22[...])
    z0[...] = quadmax(x_ref[...])
    z1[...] = quadmax(jnp.maximum(a01[...], a02[...]))
    z2[...] = quadmax(jnp.maximum(a11[...], a12[...]))
    z3[...] = quadmax(jnp.maximum(agg21, agg22))


def _final(x, b1, b2):
    (a01, a11, c21, g21) = b1
    (a02, a12, c22, g22) = b2
    zs = pl.pallas_call(
        _final_body,
        out_shape=[jax.ShapeDtypeStruct((N, C), jnp.float32)] * 4,
    )(x, a01, a11, c21, g21, a02, a12, c22, g22)
    return jnp.concatenate([z[:, ::4] for z in zs], axis=1)


def kernel(x, W1_0, b1_0, W1_1, b1_1, W1_2, b1_2,
           W2_0, b2_0, W2_1, b2_1, W2_2, b2_2):
    idx32 = _knn32(x)
    idx1 = idx32[:, :16].reshape(-1)
    idx2 = idx32[:, ::2].reshape(-1)

    bm0_1, c0_1 = _layer_mm([x], None, None, W1_0, b1_0)
    bm0_2, c0_2 = _layer_mm([x], None, None, W2_0, b2_0)
    gm0_1, gm0_2 = _gather_max2(bm0_1, idx1, bm0_2, idx2)
    bm1_1, c1_1, agg0_1 = _layer_mm([x], c0_1, gm0_1, W1_1, b1_1)
    bm1_2, c1_2, agg0_2 = _layer_mm([x], c0_2, gm0_2, W2_1, b2_1)
    gm1_1, gm1_2 = _gather_max2(bm1_1, idx1, bm1_2, idx2)
    bm2_1, c2_1, agg1_1 = _layer_mm([x, agg0_1], c1_1, gm1_1, W1_2, b1_2)
    bm2_2, c2_2, agg1_2 = _layer_mm([x, agg0_2], c1_2, gm1_2, W2_2, b2_2)
    gm2_1, gm2_2 = _gather_max2(bm2_1, idx1, bm2_2, idx2)
    return _final(x, (agg0_1, agg1_1, c2_1, gm2_1),
                  (agg0_2, agg1_2, c2_2, gm2_2))


# speedup vs baseline: 1.0930x; 1.0930x over previous
import functools

import jax
import jax.numpy as jnp
from jax import lax
from jax.experimental import pallas as pl
from jax.experimental.pallas import tpu as pltpu
from jax.experimental.pallas import tpu_sc as plsc

N = 4096
C = 128
K = 16
TM = 256


def _knn_body(sqc_ref, sqr_ref, xb_ref, xa_ref, o_ref, s_ref):
    xb = xb_ref[...]
    xa = xa_ref[...]
    xy = lax.dot_general(xb, xa, (((1,), (1,)), ((), ())),
                         preferred_element_type=jnp.float32)
    d2 = (sqc_ref[...] + sqr_ref[...]) - 2.0 * xy
    i = pl.program_id(0)
    rows = i * TM + lax.broadcasted_iota(jnp.int32, (TM, N), 0)
    cols = lax.broadcasted_iota(jnp.int32, (TM, N), 1)
    d2 = d2 + jnp.where(rows == cols, jnp.float32(1e10), jnp.float32(0.0))
    s_ref[...] = d2

    NRG = TM // 8

    def one_round(t):
        lane32 = lax.broadcasted_iota(jnp.int32, (8, 32), 1)
        for rg in range(NRG):
            sl = slice(rg * 8, rg * 8 + 8)
            S = s_ref[sl, :]
            glane = lax.broadcasted_iota(
                jnp.int32, (8, N), 1).astype(jnp.float32)
            m = jnp.min(S, axis=1, keepdims=True)
            cand = jnp.where(S == m, glane, jnp.float32(1e9))
            a = jnp.min(cand, axis=1, keepdims=True)
            o_ref[sl, :] = jnp.where(lane32 == t, a.astype(jnp.int32),
                                     o_ref[sl, :])
            s_ref[sl, :] = jnp.where(cand == a, jnp.float32(jnp.inf), S)

    def rnd(i, _):
        one_round(2 * i)
        one_round(2 * i + 1)
        return 0

    lax.fori_loop(0, 16, rnd, 0)


def _knn32(x):
    sq = jnp.sum(x * x, axis=1)
    sqc = sq.reshape(N, 1)
    sqr = sq.reshape(1, N)
    return pl.pallas_call(
        _knn_body,
        grid=(N // TM,),
        in_specs=[
            pl.BlockSpec((TM, 1), lambda i: (i, 0)),
            pl.BlockSpec((1, N), lambda i: (0, 0)),
            pl.BlockSpec((TM, C), lambda i: (i, 0)),
            pl.BlockSpec((N, C), lambda i: (0, 0)),
        ],
        out_specs=pl.BlockSpec((TM, 32), lambda i: (i, 0)),
        out_shape=jax.ShapeDtypeStruct((N, 32), jnp.int32),
        scratch_shapes=[pltpu.VMEM((TM, N), jnp.float32)],
    )(sqc, sqr, x, x)


def _leaky(z):
    return jnp.where(z >= 0, z, jnp.float32(0.2) * z)


def _layer_mm(parts, c_prev, gm_prev, W, b):
    n_parts = len(parts)
    has_agg = c_prev is not None
    cin = C * (n_parts + (1 if has_agg else 0))

    def body(*refs):
        part_refs = refs[:n_parts]
        r = n_parts
        if has_agg:
            cp_ref, gm_ref = refs[r], refs[r + 1]
            r += 2
        W_ref, b_ref = refs[r], refs[r + 1]
        outs = refs[r + 2:]
        cols = [pr[...] for pr in part_refs]
        if has_agg:
            agg = _leaky(cp_ref[...] + gm_ref[...])
            cols.append(agg)
        X = jnp.concatenate(cols, axis=1) if len(cols) > 1 else cols[0]
        Wt = W_ref[:cin, :]
        Wb = W_ref[cin:, :]
        hp = lax.Precision.HIGHEST
        bm = lax.dot_general(X, Wb, (((1,), (0,)), ((), ())), precision=hp,
                             preferred_element_type=jnp.float32)
        cc = lax.dot_general(X, Wt - Wb, (((1,), (0,)), ((), ())), precision=hp,
                             preferred_element_type=jnp.float32) + b_ref[...]
        outs[0][...] = bm
        outs[1][...] = cc
        if has_agg:
            outs[2][...] = agg

    inputs = list(parts)
    if has_agg:
        inputs += [c_prev, gm_prev]
    inputs += [W, b.reshape(1, C)]
    out_shape = [jax.ShapeDtypeStruct((N, C), jnp.float32),
                 jax.ShapeDtypeStruct((N, C), jnp.float32)]
    if has_agg:
        out_shape.append(jax.ShapeDtypeStruct((N, C), jnp.float32))
    return pl.pallas_call(body, out_shape=out_shape)(*inputs)


def _gather_max(bm, idxflat):
    NW = 32
    nodes_w = N // NW
    ch_nodes = 8
    ch_idx = ch_nodes * K
    mesh = plsc.VectorSubcoreMesh(core_axis_name="c", subcore_axis_name="s")

    nch = nodes_w // ch_nodes

    @functools.partial(
        pl.kernel, mesh=mesh,
        out_type=jax.ShapeDtypeStruct((N, C), jnp.float32),
        scratch_types=[
            pltpu.VMEM((ch_idx,), jnp.int32),
            pltpu.VMEM((ch_idx,), jnp.int32),
            pltpu.VMEM((ch_idx, C), jnp.float32),
            pltpu.VMEM((ch_idx, C), jnp.float32),
            pltpu.VMEM((ch_nodes, C), jnp.float32),
            pltpu.SemaphoreType.DMA,
            pltpu.SemaphoreType.DMA,
        ])
    def k(bm_hbm, idx_hbm, out_hbm, ia, ib, ra, rb, out_v, sa, sb):
        wid = lax.axis_index("s") * 2 + lax.axis_index("c")
        base_node = wid * nodes_w

        def start(ci, iv, rv, sem):
            pltpu.sync_copy(
                idx_hbm.at[pl.ds((base_node + ci * ch_nodes) * K, ch_idx)], iv)
            pltpu.async_copy(bm_hbm.at[iv], rv, sem)

        def finish(ci, iv, rv, sem):
            pltpu.make_async_copy(bm_hbm.at[iv], rv, sem).wait()

            @pl.loop(0, ch_nodes)
            def _node(m):
                for c in range(C // 16):
                    sl = pl.ds(c * 16, 16)
                    a0 = rv[m * K, sl]
                    a1 = rv[m * K + 1, sl]
                    for j in range(2, K, 2):
                        a0 = jnp.maximum(a0, rv[m * K + j, sl])
                        a1 = jnp.maximum(a1, rv[m * K + j + 1, sl])
                    out_v[m, sl] = jnp.maximum(a0, a1)

            pltpu.sync_copy(
                out_v, out_hbm.at[pl.ds(base_node + ci * ch_nodes, ch_nodes)])

        start(0, ia, ra, sa)

        @pl.loop(0, nch // 2 - 1)
        def _g(g):
            start(2 * g + 1, ib, rb, sb)
            finish(2 * g, ia, ra, sa)
            start(2 * g + 2, ia, ra, sa)
            finish(2 * g + 1, ib, rb, sb)

        start(nch - 1, ib, rb, sb)
        finish(nch - 2, ia, ra, sa)
        finish(nch - 1, ib, rb, sb)

    return k(bm, idxflat)


def _final_body(x_ref, a01, a11, c21, g21, a02, a12, c22, g22,
                z0, z1, z2, z3):
    def quadmax(z):
        t = jnp.maximum(z, jnp.concatenate([z[:, 1:], z[:, :1]], axis=1))
        return jnp.maximum(t, jnp.concatenate([t[:, 2:], t[:, :2]], axis=1))

    agg21 = _leaky(c21[...] + g21[...])
    agg22 = _leaky(c22[...] + g22[...])
    z0[...] = quadmax(x_ref[...])
    z1[...] = quadmax(jnp.maximum(a01[...], a02[...]))
    z2[...] = quadmax(jnp.maximum(a11[...], a12[...]))
    z3[...] = quadmax(jnp.maximum(agg21, agg22))


def _final(x, b1, b2):
    (a01, a11, c21, g21) = b1
    (a02, a12, c22, g22) = b2
    zs = pl.pallas_call(
        _final_body,
        out_shape=[jax.ShapeDtypeStruct((N, C), jnp.float32)] * 4,
    )(x, a01, a11, c21, g21, a02, a12, c22, g22)
    return jnp.concatenate([z[:, ::4] for z in zs], axis=1)


def kernel(x, W1_0, b1_0, W1_1, b1_1, W1_2, b1_2,
           W2_0, b2_0, W2_1, b2_1, W2_2, b2_2):
    idx32 = _knn32(x)
    idx1 = idx32[:, :16].reshape(-1)
    idx2 = idx32[:, ::2].reshape(-1)

    def branch(params, idxflat):
        (W0, b0), (W1, b1), (W2, b2) = params
        bm0, c0 = _layer_mm([x], None, None, W0, b0)
        gm0 = _gather_max(bm0, idxflat)
        bm1, c1, agg0 = _layer_mm([x], c0, gm0, W1, b1)
        gm1 = _gather_max(bm1, idxflat)
        bm2, c2, agg1 = _layer_mm([x, agg0], c1, gm1, W2, b2)
        gm2 = _gather_max(bm2, idxflat)
        return (agg0, agg1, c2, gm2)

    r1 = branch([(W1_0, b1_0), (W1_1, b1_1), (W1_2, b1_2)], idx1)
    r2 = branch([(W2_0, b2_0), (W2_1, b2_1), (W2_2, b2_2)], idx2)
    return _final(x, r1, r2)
